# Initial kernel scaffold; baseline (speedup 1.0000x reference)
import jax
import jax.numpy as jnp
from jax.experimental import pallas as pl


def kernel(x, emb_weight):
    raise NotImplementedError("write your pallas kernel here")


# speedup vs baseline: 1.3387x; 1.3387x over previous
import functools
import math

import jax
import jax.numpy as jnp
from jax import lax
from jax.experimental import pallas as pl
from jax.experimental.pallas import tpu as pltpu
from jax.experimental.pallas import tpu_sc as plsc

N_ROWS = 1000000
C = 32
B = 4096
L = 200

_info = plsc.get_sparse_core_info()
NC = _info.num_cores
NS = _info.num_subcores
LANES = _info.num_lanes
NW = NC * NS
B_PER_W = B // NW

SCALE = math.sqrt(C)

_mesh = plsc.VectorSubcoreMesh(core_axis_name="c", subcore_axis_name="s")


@functools.partial(
    pl.kernel,
    mesh=_mesh,
    out_type=jax.ShapeDtypeStruct((B, C * L), jnp.float32),
    compiler_params=pltpu.CompilerParams(
        needs_layout_passes=False, use_tc_tiling_on_sc=False
    ),
    scratch_types=[
        pltpu.VMEM((B_PER_W, L), jnp.int32),
        pltpu.VMEM((L, C), jnp.float32),
        pltpu.VMEM((C * L,), jnp.float32),
        pltpu.SemaphoreType.DMA,
    ],
)
def _emb_kernel(x_hbm, w_hbm, out_hbm, idx_v, rows_v, out_v, sem):
    wid = lax.axis_index("s") * NC + lax.axis_index("c")
    base = wid * B_PER_W

    pltpu.sync_copy(x_hbm.at[pl.ds(base, B_PER_W)], idx_v)

    lanes_iota = lax.iota(jnp.int32, LANES)

    def body(b, carry):
        cp1 = pltpu.async_copy(
            w_hbm.at[idx_v.at[b, pl.ds(0, 128)]],
            rows_v.at[pl.ds(0, 128)],
            sem,
        )
        cp2 = pltpu.async_copy(
            w_hbm.at[idx_v.at[b, pl.ds(128, 72)]],
            rows_v.at[pl.ds(128, 72)],
            sem,
        )
        cp1.wait()
        cp2.wait()

        def tbody(l, carry):
            for c0 in range(0, C, LANES):
                g = rows_v[l, pl.ds(c0, LANES)]
                idx_f = (lanes_iota + c0) * L + l
                plsc.store_scatter(out_v, [idx_f], g * SCALE)
            return carry

        lax.fori_loop(0, L, tbody, 0)

        pltpu.sync_copy(out_v, out_hbm.at[base + b])
        return carry

    lax.fori_loop(0, B_PER_W, body, 0)


def kernel(x, emb_weight):
    out = _emb_kernel(x.astype(jnp.int32), emb_weight)
    return out.reshape(B, C, L)
